# baseline (speedup 1.0000x reference)
import functools

import jax
import jax.numpy as jnp
from jax import lax
from jax.experimental import pallas as pl
from jax.experimental.pallas import tpu as pltpu
from jax.experimental.pallas import tpu_sc as plsc

E = 8
TOPK = 2
D = 768
DFF = 3072
R = 4
N = 4096

NB = 512
KPAD = 128

_DN_T = (((1,), (1,)), ((), ()))

_NC = 2
_NS = 16
_LANES = 16
_CH = N // (_NC * _NS)


def _logits_kernel(x_ref, gw_ref, gb_ref, out_ref):
    out_ref[...] = lax.dot_general(
        gw_ref[...], x_ref[...], _DN_T,
        preferred_element_type=jnp.float32) + gb_ref[...]


def _sc_router(logt_ref, rt_ref, buf, e1b, e2b, wb):
    f32 = jnp.float32
    wid = lax.axis_index("s") * _NC + lax.axis_index("c")
    base = wid * _CH
    pltpu.sync_copy(logt_ref.at[:, pl.ds(base, _CH)], buf)
    for j in range(_CH // _LANES):
        sl = pl.ds(j * _LANES, _LANES)
        les = [buf[e, sl] for e in range(E)]
        v1 = les[0]
        e1v = jnp.zeros((_LANES,), f32)
        v2 = jnp.full((_LANES,), -3e38, f32)
        e2v = jnp.zeros((_LANES,), f32)
        for e in range(1, E):
            le = les[e]
            ev = jnp.full((_LANES,), float(e), f32)
            gt1 = le > v1
            gt2 = le > v2
            e2v = jnp.where(gt1, e1v, jnp.where(gt2, ev, e2v))
            v2 = jnp.where(gt1, v1, jnp.where(gt2, le, v2))
            e1v = jnp.where(gt1, ev, e1v)
            v1 = jnp.where(gt1, le, v1)
        denom = jnp.exp(les[0] - v1)
        for e in range(1, E):
            denom = denom + jnp.exp(les[e] - v1)
        w = (1.0 + jnp.exp(v2 - v1)) / denom
        e1b[0, sl] = e1v
        e2b[0, sl] = e2v
        wb[0, sl] = w
    pltpu.sync_copy(e1b, rt_ref.at[pl.ds(0, 1), pl.ds(base, _CH)])
    pltpu.sync_copy(e2b, rt_ref.at[pl.ds(1, 1), pl.ds(base, _CH)])
    pltpu.sync_copy(wb, rt_ref.at[pl.ds(2, 1), pl.ds(base, _CH)])


NSPLIT = 1
DCH = DFF // NSPLIT


def _wprep_kernel(wi_ref, wo_ref, wib_ref, wob_ref):
    bf16 = jnp.bfloat16
    wib_ref[...] = wi_ref[...].astype(bf16)
    wob_ref[...] = wo_ref[...].astype(bf16)


def _ffn_kernel(x_ref, rt_ref, wib_ref, wob_ref, bo_ref,
                aT_ref, bflat_ref, out_ref):
    f32 = jnp.float32
    bf16 = jnp.bfloat16

    x = x_ref[...]
    rt = jnp.transpose(
        rt_ref[:, pl.ds(pl.program_id(0) * NB, NB)])
    e1 = rt[:, 0:1]
    e2 = rt[:, 1:2]
    w = rt[:, 2:3]

    xb = x.astype(bf16)
    U = jnp.dot(xb, aT_ref[...], preferred_element_type=f32)
    col = lax.broadcasted_iota(jnp.int32, (NB, KPAD), 1)
    ecol = (col // R).astype(f32)
    Ub = jnp.where(col == E * R, 1.0, U)
    keep1 = (ecol == e1) | (col == E * R)
    keep2 = (ecol == e2) | (col == E * R)
    S1 = jnp.where(keep1, Ub, 0.0).astype(bf16)
    S2 = jnp.where(keep2, Ub, 0.0).astype(bf16)
    S = jnp.concatenate([S1, S2], axis=0)

    dn = None
    for k in range(NSPLIT):
        ck = pl.ds(k * DCH, DCH)
        Lk = jnp.dot(S, bflat_ref[:, ck], preferred_element_type=f32)
        bk = lax.dot_general(xb, wib_ref[ck, :], _DN_T,
                             preferred_element_type=f32)
        hk = (jnp.maximum(bk + Lk[:NB], 0) +
              jnp.maximum(bk + Lk[NB:], 0)).astype(bf16)
        dnk = lax.dot_general(hk, wob_ref[:, ck], _DN_T,
                              preferred_element_type=f32)
        dn = dnk if dn is None else dn + dnk
    out_ref[...] = w * (dn + 2.0 * bo_ref[...])


def kernel(hidden_states, gate_W, gate_b, Wi, bi, Wo, bo, A, Bm):
    f32 = jnp.float32
    bf16 = jnp.bfloat16
    x = hidden_states.astype(f32)
    aT = jnp.pad(A.reshape(E * R, D).T,
                 ((0, 0), (0, KPAD - E * R))).astype(bf16)
    bflat = jnp.pad(
        jnp.concatenate([jnp.transpose(Bm, (0, 2, 1)).reshape(E * R, DFF),
                         bi[None, :]], axis=0),
        ((0, KPAD - E * R - 1), (0, 0))).astype(bf16)

    logt = pl.pallas_call(
        _logits_kernel,
        out_shape=jax.ShapeDtypeStruct((E, N), f32),
    )(x, gate_W, gate_b[:, None])

    wib, wob = pl.pallas_call(
        _wprep_kernel,
        out_shape=(jax.ShapeDtypeStruct((DFF, D), bf16),
                   jax.ShapeDtypeStruct((D, DFF), bf16)),
    )(Wi, Wo)

    mesh = plsc.VectorSubcoreMesh(core_axis_name="c", subcore_axis_name="s")
    rt = pl.kernel(
        _sc_router,
        out_type=jax.ShapeDtypeStruct((3, N), f32),
        mesh=mesh,
        scratch_types=[
            pltpu.VMEM((E, _CH), f32),
            pltpu.VMEM((1, _CH), f32),
            pltpu.VMEM((1, _CH), f32),
            pltpu.VMEM((1, _CH), f32),
        ],
    )(logt)

    out = pl.pallas_call(
        _ffn_kernel,
        grid=(N // NB,),
        in_specs=[
            pl.BlockSpec((NB, D), lambda i: (i, 0)),
            pl.BlockSpec((3, N), lambda i: (0, 0)),
            pl.BlockSpec((DFF, D), lambda i: (0, 0)),
            pl.BlockSpec((D, DFF), lambda i: (0, 0)),
            pl.BlockSpec((1, D), lambda i: (0, 0)),
            pl.BlockSpec((D, KPAD), lambda i: (0, 0)),
            pl.BlockSpec((KPAD, DFF), lambda i: (0, 0)),
        ],
        out_specs=pl.BlockSpec((NB, D), lambda i: (i, 0)),
        out_shape=jax.ShapeDtypeStruct((N, D), f32),
    )(x, rt, wib, wob, bo[None, :], aT, bflat)
    return out


# speedup vs baseline: 1.0699x; 1.0699x over previous
import functools

import jax
import jax.numpy as jnp
from jax import lax
from jax.experimental import pallas as pl
from jax.experimental.pallas import tpu as pltpu
from jax.experimental.pallas import tpu_sc as plsc

E = 8
TOPK = 2
D = 768
DFF = 3072
R = 4
N = 4096

NB = 512
KPAD = 128

_DN_T = (((1,), (1,)), ((), ()))

_NC = 2
_NS = 16
_LANES = 16
_CH = N // (_NC * _NS)


def _logits_kernel(x_ref, gw_ref, gb_ref, out_ref):
    out_ref[...] = lax.dot_general(
        gw_ref[...], x_ref[...], _DN_T,
        preferred_element_type=jnp.float32) + gb_ref[...]


def _sc_router(logt_ref, rt_ref, buf, e1b, e2b, wb):
    f32 = jnp.float32
    wid = lax.axis_index("s") * _NC + lax.axis_index("c")
    base = wid * _CH
    pltpu.sync_copy(logt_ref.at[:, pl.ds(base, _CH)], buf)
    for j in range(_CH // _LANES):
        sl = pl.ds(j * _LANES, _LANES)
        les = [buf[e, sl] for e in range(E)]
        v1 = les[0]
        e1v = jnp.zeros((_LANES,), f32)
        v2 = jnp.full((_LANES,), -3e38, f32)
        e2v = jnp.zeros((_LANES,), f32)
        for e in range(1, E):
            le = les[e]
            ev = jnp.full((_LANES,), float(e), f32)
            gt1 = le > v1
            gt2 = le > v2
            e2v = jnp.where(gt1, e1v, jnp.where(gt2, ev, e2v))
            v2 = jnp.where(gt1, v1, jnp.where(gt2, le, v2))
            e1v = jnp.where(gt1, ev, e1v)
            v1 = jnp.where(gt1, le, v1)
        denom = jnp.exp(les[0] - v1)
        for e in range(1, E):
            denom = denom + jnp.exp(les[e] - v1)
        w = (1.0 + jnp.exp(v2 - v1)) / denom
        e1b[0, sl] = e1v
        e2b[0, sl] = e2v
        wb[0, sl] = w
    pltpu.sync_copy(e1b, rt_ref.at[pl.ds(0, 1), pl.ds(base, _CH)])
    pltpu.sync_copy(e2b, rt_ref.at[pl.ds(1, 1), pl.ds(base, _CH)])
    pltpu.sync_copy(wb, rt_ref.at[pl.ds(2, 1), pl.ds(base, _CH)])


NSPLIT = 1
DCH = DFF // NSPLIT


def _ffn_kernel(x_ref, rt_ref, wi_ref, wo_ref, bo_ref,
                aT_ref, bflat_ref, out_ref, wib_ref, wob_ref):
    f32 = jnp.float32
    bf16 = jnp.bfloat16

    @pl.when(pl.program_id(0) == 0)
    def _cast_weights():
        wib_ref[...] = wi_ref[...].astype(bf16)
        wob_ref[...] = wo_ref[...].astype(bf16)

    x = x_ref[...]
    rt = jnp.transpose(
        rt_ref[:, pl.ds(pl.program_id(0) * NB, NB)])
    e1 = rt[:, 0:1]
    e2 = rt[:, 1:2]
    w = rt[:, 2:3]

    xb = x.astype(bf16)
    U = jnp.dot(xb, aT_ref[...], preferred_element_type=f32)
    col = lax.broadcasted_iota(jnp.int32, (NB, KPAD), 1)
    ecol = (col // R).astype(f32)
    Ub = jnp.where(col == E * R, 1.0, U)
    keep1 = (ecol == e1) | (col == E * R)
    keep2 = (ecol == e2) | (col == E * R)
    S1 = jnp.where(keep1, Ub, 0.0).astype(bf16)
    S2 = jnp.where(keep2, Ub, 0.0).astype(bf16)
    S = jnp.concatenate([S1, S2], axis=0)

    dn = None
    for k in range(NSPLIT):
        ck = pl.ds(k * DCH, DCH)
        Lk = jnp.dot(S, bflat_ref[:, ck], preferred_element_type=f32)
        bk = lax.dot_general(xb, wib_ref[ck, :], _DN_T,
                             preferred_element_type=f32)
        hk = (jnp.maximum(bk + Lk[:NB], 0) +
              jnp.maximum(bk + Lk[NB:], 0)).astype(bf16)
        dnk = lax.dot_general(hk, wob_ref[:, ck], _DN_T,
                              preferred_element_type=f32)
        dn = dnk if dn is None else dn + dnk
    out_ref[...] = w * (dn + 2.0 * bo_ref[...])


def kernel(hidden_states, gate_W, gate_b, Wi, bi, Wo, bo, A, Bm):
    f32 = jnp.float32
    bf16 = jnp.bfloat16
    x = hidden_states.astype(f32)
    aT = jnp.pad(A.reshape(E * R, D).T,
                 ((0, 0), (0, KPAD - E * R))).astype(bf16)
    bflat = jnp.pad(
        jnp.concatenate([jnp.transpose(Bm, (0, 2, 1)).reshape(E * R, DFF),
                         bi[None, :]], axis=0),
        ((0, KPAD - E * R - 1), (0, 0))).astype(bf16)

    logt = pl.pallas_call(
        _logits_kernel,
        out_shape=jax.ShapeDtypeStruct((E, N), f32),
    )(x, gate_W, gate_b[:, None])

    mesh = plsc.VectorSubcoreMesh(core_axis_name="c", subcore_axis_name="s")
    rt = pl.kernel(
        _sc_router,
        out_type=jax.ShapeDtypeStruct((3, N), f32),
        mesh=mesh,
        scratch_types=[
            pltpu.VMEM((E, _CH), f32),
            pltpu.VMEM((1, _CH), f32),
            pltpu.VMEM((1, _CH), f32),
            pltpu.VMEM((1, _CH), f32),
        ],
    )(logt)

    out = pl.pallas_call(
        _ffn_kernel,
        grid=(N // NB,),
        in_specs=[
            pl.BlockSpec((NB, D), lambda i: (i, 0)),
            pl.BlockSpec((3, N), lambda i: (0, 0)),
            pl.BlockSpec((DFF, D), lambda i: (0, 0)),
            pl.BlockSpec((D, DFF), lambda i: (0, 0)),
            pl.BlockSpec((1, D), lambda i: (0, 0)),
            pl.BlockSpec((D, KPAD), lambda i: (0, 0)),
            pl.BlockSpec((KPAD, DFF), lambda i: (0, 0)),
        ],
        out_specs=pl.BlockSpec((NB, D), lambda i: (i, 0)),
        out_shape=jax.ShapeDtypeStruct((N, D), f32),
        scratch_shapes=[
            pltpu.VMEM((DFF, D), bf16),
            pltpu.VMEM((D, DFF), bf16),
        ],
    )(x, rt, Wi, Wo, bo[None, :], aT, bflat)
    return out
